# baseline (speedup 1.0000x reference)
import dataclasses
import functools

import jax
import jax.numpy as jnp
from jax import lax
from jax.experimental import pallas as pl
from jax.experimental.pallas import tpu as pltpu
from jax.experimental.pallas import tpu_sc as plsc

NU = 50000
NI = 50000
NNZ = 1600000
D = 64
DH = D // 2

LANES = 128
SUBS = 3
CHUNK = LANES * SUBS
NSUB = 16
NROWS_P = 12576
CHUNKS_PER_TILE = NROWS_P // (NSUB * SUBS)
MAIN_T = CHUNKS_PER_TILE - 2
TILE_ROWS = 3128
ACC_ROWS = TILE_ROWS * NSUB
LAST_ROWS = NU - (NSUB - 1) * TILE_ROWS

_mesh = plsc.VectorSubcoreMesh(core_axis_name="c", subcore_axis_name="s")

_cp = pltpu.CompilerParams()
if "needs_layout_passes" in pltpu.CompilerParams.__dataclass_fields__:
    _cp = dataclasses.replace(_cp, needs_layout_passes=False)
if "use_tc_tiling_on_sc" in pltpu.CompilerParams.__dataclass_fields__:
    _cp = dataclasses.replace(_cp, use_tc_tiling_on_sc=False)


def _run_direction(ebufs, ldsts, rowss, lsems, gsems, ssems, acc,
                   table_hbm, out_hbm, zeros_hbm, pack_hbm, gslot, dslot,
                   c, s):
    pltpu.sync_copy(zeros_hbm, acc.at[pl.ds(s * TILE_ROWS, TILE_ROWS)])
    plsc.subcore_barrier()

    def fire_load(t, p):
        r = SUBS * s + t * (NSUB * SUBS)
        return pltpu.async_copy(pack_hbm.at[pl.ds(3 * r, 3 * SUBS)],
                                ebufs[p], lsems[p])

    def wait_load(p):
        pltpu.make_async_copy(pack_hbm.at[pl.ds(0, 3 * SUBS)], ebufs[p],
                              lsems[p]).wait()

    def fire_gather(p):
        for j in range(SUBS):
            pltpu.async_copy(table_hbm.at[ebufs[p].at[gslot + 3 * j]],
                             rowss[p].at[pl.ds(j * LANES, LANES)], gsems[p])

    def wait_gather(p):
        for j in range(SUBS):
            pltpu.make_async_copy(
                table_hbm.at[ebufs[p].at[gslot + 3 * j]],
                rowss[p].at[pl.ds(j * LANES, LANES)], gsems[p]).wait()

    def fire_scatter(p):
        for j in range(SUBS):
            pltpu.async_copy(rowss[p].at[pl.ds(j * LANES, LANES)],
                             acc.at[ldsts[p].at[j]], ssems[p], add=True)

    def wait_scatter(p):
        for j in range(SUBS):
            pltpu.make_async_copy(rowss[p].at[pl.ds(j * LANES, LANES)],
                                  acc.at[ldsts[p].at[j]], ssems[p]).wait()

    def fire_scatter_sub(p, j):
        pltpu.async_copy(rowss[p].at[pl.ds(j * LANES, LANES)],
                         acc.at[ldsts[p].at[j]], ssems[p], add=True)

    def compute_and_scatter(p):
        ebuf, ldst, rows = ebufs[p], ldsts[p], rowss[p]
        for j in range(SUBS):
            @pl.loop(0, LANES, step=16)
            def _grp(k16):
                sl16 = pl.ds(k16, 16)
                ldst[j, sl16] = ebuf[dslot + 3 * j, sl16]
                v16 = plsc.bitcast(ebuf[2 + 3 * j, sl16], jnp.float32)
                for e in range(16):
                    sp = lax.gather(
                        v16, jnp.full((16, 1), e, jnp.int32),
                        lax.GatherDimensionNumbers(
                            offset_dims=(), collapsed_slice_dims=(0,),
                            start_index_map=(0,)),
                        (1,), mode=lax.GatherScatterMode.PROMISE_IN_BOUNDS)
                    for q in range(DH // 16):
                        sl = pl.ds(q * 16, 16)
                        r = j * LANES + k16 + e
                        rows[r, sl] = rows[r, sl] * sp
            fire_scatter_sub(p, j)

    fire_load(0, 0)
    wait_load(0)
    fire_gather(0)
    fire_load(1, 1)

    @pl.loop(0, MAIN_T // 2)
    def _outer(k):
        for p in (0, 1):
            t = 2 * k + p
            wait_gather(p)
            wait_load(1 - p)
            if p == 0:
                @pl.when(k >= 1)
                def _():
                    wait_scatter(1 - p)
            else:
                wait_scatter(1 - p)
            fire_gather(1 - p)
            compute_and_scatter(p)
            fire_load(t + 2, p)

    wait_gather(0)
    wait_load(1)
    wait_scatter(1)
    fire_gather(1)
    compute_and_scatter(0)

    wait_gather(1)
    wait_scatter(0)
    compute_and_scatter(1)
    wait_scatter(1)

    plsc.subcore_barrier()

    @pl.when(s < NSUB - 1)
    def _():
        pltpu.sync_copy(acc.at[pl.ds(s * TILE_ROWS, TILE_ROWS)],
                        out_hbm.at[c, pl.ds(s * TILE_ROWS, TILE_ROWS)])

    @pl.when(s == NSUB - 1)
    def _():
        pltpu.sync_copy(
            acc.at[pl.ds((NSUB - 1) * TILE_ROWS, LAST_ROWS)],
            out_hbm.at[c, pl.ds((NSUB - 1) * TILE_ROWS, LAST_ROWS)])

    plsc.subcore_barrier()


@functools.partial(
    pl.kernel,
    out_type=(jax.ShapeDtypeStruct((2, NU, DH), jnp.float32),
              jax.ShapeDtypeStruct((2, NI, DH), jnp.float32)),
    mesh=_mesh,
    scratch_types=(
        [pltpu.VMEM((3 * SUBS, LANES), jnp.int32)] * 2
        + [pltpu.VMEM((SUBS, LANES), jnp.int32)] * 2
        + [pltpu.VMEM((CHUNK, DH), jnp.float32)] * 2
        + [pltpu.VMEM_SHARED((ACC_ROWS, DH), jnp.float32)]
        + [pltpu.SemaphoreType.DMA] * 6
    ),
    compiler_params=_cp,
)
def _lightgcn_sc(user_hbm, item_hbm, pack_hbm, zeros_hbm, uout_hbm, iout_hbm,
                 *refs):
    c = lax.axis_index("c")
    s = lax.axis_index("s")
    ebufs = refs[0:2]
    ldsts = refs[2:4]
    rowss = refs[4:6]
    acc = refs[6]
    lsems = refs[7:9]
    gsems = refs[9:11]
    ssems = refs[11:13]
    _run_direction(ebufs, ldsts, rowss, lsems, gsems, ssems, acc,
                   item_hbm.at[c], uout_hbm, zeros_hbm, pack_hbm, 1, 0, c, s)
    _run_direction(ebufs, ldsts, rowss, lsems, gsems, ssems, acc,
                   user_hbm.at[c], iout_hbm, zeros_hbm, pack_hbm, 0, 1, c, s)


def kernel(user_emb, item_emb, row_idx, col_idx, values):
    pad = NROWS_P * LANES - NNZ
    pad_idx = (jnp.arange(pad, dtype=jnp.int32) * 16) % NU
    row_p = jnp.concatenate([row_idx.astype(jnp.int32), pad_idx])
    col_p = jnp.concatenate([col_idx.astype(jnp.int32), pad_idx])
    val_p = jnp.concatenate(
        [lax.bitcast_convert_type(values, jnp.int32),
         jnp.zeros((pad,), jnp.int32)])
    pack = jnp.stack([row_p.reshape(NROWS_P, LANES),
                      col_p.reshape(NROWS_P, LANES),
                      val_p.reshape(NROWS_P, LANES)], axis=1)
    pack = pack.reshape(3 * NROWS_P, LANES)
    user_h = jnp.stack([user_emb[:, :DH], user_emb[:, DH:]])
    item_h = jnp.stack([item_emb[:, :DH], item_emb[:, DH:]])
    zeros = jnp.zeros((TILE_ROWS, DH), jnp.float32)
    u_h, i_h = _lightgcn_sc(user_h, item_h, pack, zeros)
    user_new = jnp.concatenate([u_h[0], u_h[1]], axis=1)
    item_new = jnp.concatenate([i_h[0], i_h[1]], axis=1)
    return user_new, item_new


# speedup vs baseline: 1.0013x; 1.0013x over previous
import dataclasses
import functools

import jax
import jax.numpy as jnp
from jax import lax
from jax.experimental import pallas as pl
from jax.experimental.pallas import tpu as pltpu
from jax.experimental.pallas import tpu_sc as plsc

NU = 50000
NI = 50000
NNZ = 1600000
D = 64
DH = D // 2

LANES = 128
SUBS = 3
CHUNK = LANES * SUBS
NSUB = 16
NROWS_P = 12576
CHUNKS_PER_TILE = NROWS_P // (NSUB * SUBS)
MAIN_T = CHUNKS_PER_TILE - 2
TILE_ROWS = 3128
ACC_ROWS = TILE_ROWS * NSUB
LAST_ROWS = NU - (NSUB - 1) * TILE_ROWS

_mesh = plsc.VectorSubcoreMesh(core_axis_name="c", subcore_axis_name="s")

_cp = pltpu.CompilerParams()
if "needs_layout_passes" in pltpu.CompilerParams.__dataclass_fields__:
    _cp = dataclasses.replace(_cp, needs_layout_passes=False)
if "use_tc_tiling_on_sc" in pltpu.CompilerParams.__dataclass_fields__:
    _cp = dataclasses.replace(_cp, use_tc_tiling_on_sc=False)


def _run_direction(ebufs, ldsts, rowss, lsems, gsems, ssems, acc,
                   table_hbm, out_hbm, zeros_hbm, pack_hbm, gslot, dslot,
                   c, s):
    pltpu.sync_copy(zeros_hbm, acc.at[pl.ds(s * TILE_ROWS, TILE_ROWS)])
    plsc.subcore_barrier()

    def fire_load(t, p):
        r = SUBS * s + t * (NSUB * SUBS)
        return pltpu.async_copy(pack_hbm.at[pl.ds(3 * r, 3 * SUBS)],
                                ebufs[p], lsems[p])

    def wait_load(p):
        pltpu.make_async_copy(pack_hbm.at[pl.ds(0, 3 * SUBS)], ebufs[p],
                              lsems[p]).wait()

    def fire_gather(p):
        for j in range(SUBS):
            pltpu.async_copy(table_hbm.at[ebufs[p].at[gslot + 3 * j]],
                             rowss[p].at[pl.ds(j * LANES, LANES)], gsems[p])

    def wait_gather(p):
        for j in range(SUBS):
            pltpu.make_async_copy(
                table_hbm.at[ebufs[p].at[gslot + 3 * j]],
                rowss[p].at[pl.ds(j * LANES, LANES)], gsems[p]).wait()

    def wait_scatter(p):
        for j in range(SUBS):
            pltpu.make_async_copy(rowss[p].at[pl.ds(j * LANES, LANES)],
                                  acc.at[ldsts[p].at[j]], ssems[p]).wait()

    def fire_scatter_sub(p, j):
        pltpu.async_copy(rowss[p].at[pl.ds(j * LANES, LANES)],
                         acc.at[ldsts[p].at[j]], ssems[p], add=True)

    def compute_and_scatter(p):
        ebuf, ldst, rows = ebufs[p], ldsts[p], rowss[p]
        for j in range(SUBS):
            @pl.loop(0, LANES, step=16)
            def _grp(k16):
                sl16 = pl.ds(k16, 16)
                ldst[j, sl16] = ebuf[dslot + 3 * j, sl16]
                v16 = plsc.bitcast(ebuf[2 + 3 * j, sl16], jnp.float32)
                for e in range(16):
                    sp = v16[e]
                    for q in range(DH // 16):
                        sl = pl.ds(q * 16, 16)
                        r = j * LANES + k16 + e
                        rows[r, sl] = rows[r, sl] * sp
            fire_scatter_sub(p, j)

    fire_load(0, 0)
    wait_load(0)
    fire_gather(0)
    fire_load(1, 1)

    @pl.loop(0, MAIN_T // 2)
    def _outer(k):
        for p in (0, 1):
            t = 2 * k + p
            wait_gather(p)
            wait_load(1 - p)
            if p == 0:
                @pl.when(k >= 1)
                def _():
                    wait_scatter(1 - p)
            else:
                wait_scatter(1 - p)
            fire_gather(1 - p)
            compute_and_scatter(p)
            fire_load(t + 2, p)

    wait_gather(0)
    wait_load(1)
    wait_scatter(1)
    fire_gather(1)
    compute_and_scatter(0)

    wait_gather(1)
    wait_scatter(0)
    compute_and_scatter(1)
    wait_scatter(1)

    plsc.subcore_barrier()

    @pl.when(s < NSUB - 1)
    def _():
        pltpu.sync_copy(acc.at[pl.ds(s * TILE_ROWS, TILE_ROWS)],
                        out_hbm.at[c, pl.ds(s * TILE_ROWS, TILE_ROWS)])

    @pl.when(s == NSUB - 1)
    def _():
        pltpu.sync_copy(
            acc.at[pl.ds((NSUB - 1) * TILE_ROWS, LAST_ROWS)],
            out_hbm.at[c, pl.ds((NSUB - 1) * TILE_ROWS, LAST_ROWS)])

    plsc.subcore_barrier()


@functools.partial(
    pl.kernel,
    out_type=(jax.ShapeDtypeStruct((2, NU, DH), jnp.float32),
              jax.ShapeDtypeStruct((2, NI, DH), jnp.float32)),
    mesh=_mesh,
    scratch_types=(
        [pltpu.VMEM((3 * SUBS, LANES), jnp.int32)] * 2
        + [pltpu.VMEM((SUBS, LANES), jnp.int32)] * 2
        + [pltpu.VMEM((CHUNK, DH), jnp.float32)] * 2
        + [pltpu.VMEM_SHARED((ACC_ROWS, DH), jnp.float32)]
        + [pltpu.SemaphoreType.DMA] * 6
    ),
    compiler_params=_cp,
)
def _lightgcn_sc(user_hbm, item_hbm, pack_hbm, zeros_hbm, uout_hbm, iout_hbm,
                 *refs):
    c = lax.axis_index("c")
    s = lax.axis_index("s")
    ebufs = refs[0:2]
    ldsts = refs[2:4]
    rowss = refs[4:6]
    acc = refs[6]
    lsems = refs[7:9]
    gsems = refs[9:11]
    ssems = refs[11:13]
    _run_direction(ebufs, ldsts, rowss, lsems, gsems, ssems, acc,
                   item_hbm.at[c], uout_hbm, zeros_hbm, pack_hbm, 1, 0, c, s)
    _run_direction(ebufs, ldsts, rowss, lsems, gsems, ssems, acc,
                   user_hbm.at[c], iout_hbm, zeros_hbm, pack_hbm, 0, 1, c, s)


def kernel(user_emb, item_emb, row_idx, col_idx, values):
    pad = NROWS_P * LANES - NNZ
    pad_idx = (jnp.arange(pad, dtype=jnp.int32) * 16) % NU
    row_p = jnp.concatenate([row_idx.astype(jnp.int32), pad_idx])
    col_p = jnp.concatenate([col_idx.astype(jnp.int32), pad_idx])
    val_p = jnp.concatenate(
        [lax.bitcast_convert_type(values, jnp.int32),
         jnp.zeros((pad,), jnp.int32)])
    pack = jnp.stack([row_p.reshape(NROWS_P, LANES),
                      col_p.reshape(NROWS_P, LANES),
                      val_p.reshape(NROWS_P, LANES)], axis=1)
    pack = pack.reshape(3 * NROWS_P, LANES)
    user_h = jnp.stack([user_emb[:, :DH], user_emb[:, DH:]])
    item_h = jnp.stack([item_emb[:, :DH], item_emb[:, DH:]])
    zeros = jnp.zeros((TILE_ROWS, DH), jnp.float32)
    u_h, i_h = _lightgcn_sc(user_h, item_h, pack, zeros)
    user_new = jnp.concatenate([u_h[0], u_h[1]], axis=1)
    item_new = jnp.concatenate([i_h[0], i_h[1]], axis=1)
    return user_new, item_new
